# baseline (speedup 1.0000x reference)
import functools

import jax
import jax.numpy as jnp
from jax import lax
from jax.experimental import pallas as pl
from jax.experimental.pallas import tpu as pltpu
from jax.experimental.pallas import tpu_sc as plsc

V = 1000000
D = 64
LABELS = 2
B = 4096
L = 200

NW = 32
BPW = B // NW
LH = L // 2

_mesh = plsc.VectorSubcoreMesh(core_axis_name="c", subcore_axis_name="s")


@functools.partial(
    pl.kernel,
    out_type=jax.ShapeDtypeStruct((B * LABELS,), jnp.float32),
    mesh=_mesh,
    compiler_params=pltpu.CompilerParams(
        needs_layout_passes=False, use_tc_tiling_on_sc=False),
    scratch_types=[
        pltpu.VMEM((2, LH), jnp.int32),
        pltpu.VMEM((LH, D), jnp.float32),
        pltpu.VMEM((LH, D), jnp.float32),
        pltpu.VMEM((BPW + 16,), jnp.float32),
        pltpu.VMEM((LABELS, D), jnp.float32),
        pltpu.VMEM((16,), jnp.float32),
        pltpu.VMEM((BPW * 16,), jnp.float32),
        pltpu.VMEM((BPW * LABELS,), jnp.float32),
        pltpu.SemaphoreType.DMA,
    ],
)
def _embd_sc_kernel(x_hbm, xlen_hbm, table_hbm, w_hbm, b_hbm, out_hbm,
                    idx_v, rows_a, rows_b, xlen_v, w_v, b_v, tmp_v, out_v,
                    sem):
    wid = lax.axis_index("s") * 2 + lax.axis_index("c")
    base = wid * BPW

    pltpu.sync_copy(xlen_hbm.at[pl.ds(base, BPW)], xlen_v.at[pl.ds(0, BPW)])
    pltpu.sync_copy(w_hbm, w_v)
    pltpu.sync_copy(b_hbm, b_v)

    bvec = b_v[...]
    b0 = bvec[0]
    b1 = bvec[1]
    w00 = w_v[0, pl.ds(0, 16)]
    w01 = w_v[0, pl.ds(16, 16)]
    w02 = w_v[0, pl.ds(32, 16)]
    w03 = w_v[0, pl.ds(48, 16)]
    w10 = w_v[1, pl.ds(0, 16)]
    w11 = w_v[1, pl.ds(16, 16)]
    w12 = w_v[1, pl.ds(32, 16)]
    w13 = w_v[1, pl.ds(48, 16)]
    lane = lax.iota(jnp.int32, 16)

    def row_body(r, _):
        pltpu.sync_copy(x_hbm.at[base + r], idx_v)
        cp_a = pltpu.async_copy(table_hbm.at[idx_v.at[0]], rows_a, sem)
        cp_b = pltpu.async_copy(table_hbm.at[idx_v.at[1]], rows_b, sem)
        cp_a.wait()
        cp_b.wait()

        def red(i, accs):
            a0, a1, a2, a3 = accs
            a0 = a0 + rows_a[i, pl.ds(0, 16)] + rows_b[i, pl.ds(0, 16)]
            a1 = a1 + rows_a[i, pl.ds(16, 16)] + rows_b[i, pl.ds(16, 16)]
            a2 = a2 + rows_a[i, pl.ds(32, 16)] + rows_b[i, pl.ds(32, 16)]
            a3 = a3 + rows_a[i, pl.ds(48, 16)] + rows_b[i, pl.ds(48, 16)]
            return (a0, a1, a2, a3)

        z = jnp.zeros((16,), jnp.float32)
        a0, a1, a2, a3 = lax.fori_loop(0, LH, red, (z, z, z, z))

        inv = (1.0 / xlen_v[pl.ds(r, 16)])[0]
        d0 = a0 * w00 + a1 * w01 + a2 * w02 + a3 * w03
        d1 = a0 * w10 + a1 * w11 + a2 * w12 + a3 * w13
        s0 = jnp.sum(d0) * inv + b0
        s1 = jnp.sum(d1) * inv + b1
        vres = jnp.where(lane == 0, jnp.full((16,), s0, jnp.float32),
                         jnp.full((16,), s1, jnp.float32))
        vres = 1.0 / (1.0 + jnp.exp(-vres))
        tmp_v[pl.ds(r * 16, 16)] = vres
        return 0

    lax.fori_loop(0, BPW, row_body, 0)

    def pack_body(g, _):
        idx = ((lane >> 1) + g * 8) * 16 + (lane & 1)
        out_v[pl.ds(g * 16, 16)] = plsc.load_gather(tmp_v, [idx])
        return 0

    lax.fori_loop(0, (BPW * LABELS) // 16, pack_body, 0)

    pltpu.sync_copy(out_v, out_hbm.at[pl.ds(base * LABELS, BPW * LABELS)])


def kernel(x, xlength, embd_table, fc_W, fc_b):
    x3 = x.reshape(B, 2, LH)
    xlen_flat = xlength.reshape(B)
    b_pad = jnp.zeros((16,), jnp.float32).at[:LABELS].set(fc_b)
    out_flat = _embd_sc_kernel(x3, xlen_flat, embd_table, fc_W, b_pad)
    return out_flat.reshape(B, LABELS)


# speedup vs baseline: 1.2114x; 1.2114x over previous
import functools

import jax
import jax.numpy as jnp
from jax import lax
from jax.experimental import pallas as pl
from jax.experimental.pallas import tpu as pltpu
from jax.experimental.pallas import tpu_sc as plsc

V = 1000000
D = 64
LABELS = 2
B = 4096
L = 200

NW = 32
BPW = B // NW
LH = L // 2

_mesh = plsc.VectorSubcoreMesh(core_axis_name="c", subcore_axis_name="s")


@functools.partial(
    pl.kernel,
    out_type=jax.ShapeDtypeStruct((B * LABELS,), jnp.float32),
    mesh=_mesh,
    compiler_params=pltpu.CompilerParams(
        needs_layout_passes=False, use_tc_tiling_on_sc=False),
    scratch_types=[
        pltpu.VMEM((2, 2, LH), jnp.int32),
        pltpu.VMEM((2, 2, LH, D), jnp.float32),
        pltpu.VMEM((BPW + 16,), jnp.float32),
        pltpu.VMEM((LABELS, D), jnp.float32),
        pltpu.VMEM((16,), jnp.float32),
        pltpu.VMEM((BPW * 16,), jnp.float32),
        pltpu.VMEM((BPW * LABELS,), jnp.float32),
        pltpu.SemaphoreType.DMA,
        pltpu.SemaphoreType.DMA,
        pltpu.SemaphoreType.DMA,
        pltpu.SemaphoreType.DMA,
    ],
)
def _embd_sc_kernel(x_hbm, xlen_hbm, table_hbm, w_hbm, b_hbm, out_hbm,
                    idx_v, rows_v, xlen_v, w_v, b_v, tmp_v, out_v,
                    gsem0, gsem1, isem0, isem1):
    wid = lax.axis_index("s") * 2 + lax.axis_index("c")
    base = wid * BPW

    pltpu.sync_copy(xlen_hbm.at[pl.ds(base, BPW)], xlen_v.at[pl.ds(0, BPW)])
    pltpu.sync_copy(w_hbm, w_v)
    pltpu.sync_copy(b_hbm, b_v)

    gsem = (gsem0, gsem1)
    isem = (isem0, isem1)

    bvec = b_v[...]
    b0 = bvec[0]
    b1 = bvec[1]
    w00 = w_v[0, pl.ds(0, 16)]
    w01 = w_v[0, pl.ds(16, 16)]
    w02 = w_v[0, pl.ds(32, 16)]
    w03 = w_v[0, pl.ds(48, 16)]
    w10 = w_v[1, pl.ds(0, 16)]
    w11 = w_v[1, pl.ds(16, 16)]
    w12 = w_v[1, pl.ds(32, 16)]
    w13 = w_v[1, pl.ds(48, 16)]
    lane = lax.iota(jnp.int32, 16)

    def idx_copy(r, slot):
        r = jnp.minimum(r, BPW - 1)
        return pltpu.make_async_copy(
            x_hbm.at[base + r], idx_v.at[slot], isem[slot])

    def gather_copy(slot, h):
        return pltpu.make_async_copy(
            table_hbm.at[idx_v.at[slot, h]], rows_v.at[slot, h], gsem[slot])

    def gather_start(slot):
        gather_copy(slot, 0).start()
        gather_copy(slot, 1).start()

    def gather_wait(slot):
        gather_copy(slot, 0).wait()
        gather_copy(slot, 1).wait()

    def reduce_row(slot, r):
        def red(i, accs):
            a0, a1, a2, a3 = accs
            a0 = (a0 + rows_v[slot, 0, i, pl.ds(0, 16)]
                  + rows_v[slot, 1, i, pl.ds(0, 16)])
            a1 = (a1 + rows_v[slot, 0, i, pl.ds(16, 16)]
                  + rows_v[slot, 1, i, pl.ds(16, 16)])
            a2 = (a2 + rows_v[slot, 0, i, pl.ds(32, 16)]
                  + rows_v[slot, 1, i, pl.ds(32, 16)])
            a3 = (a3 + rows_v[slot, 0, i, pl.ds(48, 16)]
                  + rows_v[slot, 1, i, pl.ds(48, 16)])
            return (a0, a1, a2, a3)

        z = jnp.zeros((16,), jnp.float32)
        a0, a1, a2, a3 = lax.fori_loop(0, LH, red, (z, z, z, z), unroll=5)

        inv = (1.0 / xlen_v[pl.ds(r, 16)])[0]
        d0 = a0 * w00 + a1 * w01 + a2 * w02 + a3 * w03
        d1 = a0 * w10 + a1 * w11 + a2 * w12 + a3 * w13
        s0 = jnp.sum(d0) * inv + b0
        s1 = jnp.sum(d1) * inv + b1
        vres = jnp.where(lane == 0, jnp.full((16,), s0, jnp.float32),
                         jnp.full((16,), s1, jnp.float32))
        vres = 1.0 / (1.0 + jnp.exp(-vres))
        tmp_v[pl.ds(r * 16, 16)] = vres

    idx_copy(0, 0).start()
    idx_copy(1, 1).start()
    idx_copy(0, 0).wait()
    gather_start(0)
    idx_copy(1, 1).wait()
    gather_start(1)

    def pair_body(rp, _):
        r0 = 2 * rp
        gather_wait(0)
        idx_copy(r0 + 2, 0).start()
        reduce_row(0, r0)
        gather_wait(1)
        idx_copy(r0 + 3, 1).start()
        idx_copy(r0 + 2, 0).wait()
        gather_start(0)
        reduce_row(1, r0 + 1)
        idx_copy(r0 + 3, 1).wait()
        gather_start(1)
        return 0

    lax.fori_loop(0, BPW // 2, pair_body, 0)

    gather_wait(0)
    gather_wait(1)

    def pack_body(g, _):
        idx = ((lane >> 1) + g * 8) * 16 + (lane & 1)
        out_v[pl.ds(g * 16, 16)] = plsc.load_gather(tmp_v, [idx])
        return 0

    lax.fori_loop(0, (BPW * LABELS) // 16, pack_body, 0)

    pltpu.sync_copy(out_v, out_hbm.at[pl.ds(base * LABELS, BPW * LABELS)])


def kernel(x, xlength, embd_table, fc_W, fc_b):
    x3 = x.reshape(B, 2, LH)
    xlen_flat = xlength.reshape(B)
    b_pad = jnp.zeros((16,), jnp.float32).at[:LABELS].set(fc_b)
    out_flat = _embd_sc_kernel(x3, xlen_flat, embd_table, fc_W, b_pad)
    return out_flat.reshape(B, LABELS)
